# baseline (speedup 1.0000x reference)
import jax
import jax.numpy as jnp
from jax import lax
from jax.experimental import pallas as pl
from jax.experimental.pallas import tpu as pltpu

_INPUT_DIM = 768
_SPARSE_DIM = 32768
_BATCH = 128
_K = 64

_BLK = 1024
_NB = _SPARSE_DIM // _BLK

_KEY_POS_ZERO = 0x80000000
_KEY_NEG_ZERO = 0x7FFFFFFF


def _ukey(v):
    k = lax.bitcast_convert_type(v, jnp.int32)
    k2 = jnp.where(k < 0, jnp.bitwise_xor(k, jnp.int32(0x7FFFFFFF)), k)
    return lax.bitcast_convert_type(k2, jnp.uint32) + jnp.uint32(_KEY_POS_ZERO)


def _unkey(u):
    k2 = lax.bitcast_convert_type(u - jnp.uint32(_KEY_POS_ZERO), jnp.int32)
    k = jnp.where(k2 < 0, jnp.bitwise_xor(k2, jnp.int32(0x7FFFFFFF)), k2)
    return lax.bitcast_convert_type(k, jnp.float32)


def _count_dot(mask, ones_bf):
    mb = jnp.where(mask, jnp.bfloat16(1.0), jnp.bfloat16(0.0))
    res = lax.dot_general(mb, ones_bf, (((1,), (0,)), ((), ())),
                          preferred_element_type=jnp.float32)
    return res[:, 0:1]


def _s16(raw):
    return (raw - 32768).astype(jnp.int16)


def _bisect16(arr_ref, ones_bf, kk, per_row):
    if per_row:
        init = jnp.zeros((_BATCH, 1), jnp.int32)
    else:
        init = jnp.int32(0)

    def body(t, T):
        cand = T | (jnp.int32(1) << (15 - t))
        cnt = _count_dot(arr_ref[...] >= _s16(cand), ones_bf)
        if not per_row:
            cnt = jnp.sum(cnt)
        return jnp.where(cnt >= kk, cand, T)

    return lax.fori_loop(0, 16, body, init)


def _sae_kernel(x_ref, w_ref, bias_ref, la_ref, recon_ref, dead_ref,
                khi_ref, klo_ref, khi2_ref, mlo_ref, xb_ref, tg_ref, tb_ref):
    i = pl.program_id(0)
    ones_bf = jnp.full((_SPARSE_DIM, 8), jnp.bfloat16(1.0))

    @pl.when(i == 0)
    def _():
        xb_ref[...] = x_ref[...] - bias_ref[...]

    @pl.when(i < _NB)
    def _():
        lin_blk = lax.dot_general(
            xb_ref[...], w_ref[...],
            (((1,), (1,)), ((), ())),
            preferred_element_type=jnp.float32)
        u = lax.bitcast_convert_type(_ukey(lin_blk), jnp.int32)
        sl = pl.ds(i * _BLK, _BLK)
        khi_ref[:, sl] = _s16(jnp.bitwise_and(u >> 16, jnp.int32(0xFFFF)))
        klo_ref[:, sl] = _s16(jnp.bitwise_and(u, jnp.int32(0xFFFF)))

    @pl.when(i == _NB - 1)
    def _():
        kk_g = jnp.float32(_K * _BATCH)

        tg_hi = _bisect16(khi_ref, ones_bf, kk_g, per_row=False)
        tg_hi16 = _s16(tg_hi)
        base_g = jnp.sum(_count_dot(khi_ref[...] > tg_hi16, ones_bf))
        mlo_ref[...] = jnp.where(khi_ref[...] == tg_hi16, klo_ref[...],
                                 jnp.int16(-32768))
        tg_lo = _bisect16(mlo_ref, ones_bf, kk_g - base_g, per_row=False)
        tg32 = lax.bitcast_convert_type((tg_hi << 16) | tg_lo, jnp.uint32)
        tg_ref[0] = tg32

        rawhi = khi_ref[...].astype(jnp.int32) + 32768
        rawlo = klo_ref[...].astype(jnp.int32) + 32768
        key32 = lax.bitcast_convert_type((rawhi << 16) | rawlo, jnp.uint32)
        sel = ((key32 >= tg32) & (key32 != jnp.uint32(_KEY_POS_ZERO))
               & (key32 != jnp.uint32(_KEY_NEG_ZERO)))
        nd = jnp.max(sel.astype(jnp.int32), axis=0, keepdims=True)
        dead = ((la_ref[...] + 1.0) * (1.0 - nd.astype(jnp.float32))) > 0.0
        khi2_ref[...] = jnp.where(dead, khi_ref[...], jnp.int16(-32768))

        kk_r = jnp.float32(2 * _K)
        tb_hi = _bisect16(khi2_ref, ones_bf, kk_r, per_row=True)
        tb_hi16 = _s16(tb_hi)
        base_b = _count_dot(khi2_ref[...] > tb_hi16, ones_bf)
        mlo_ref[...] = jnp.where(khi2_ref[...] == tb_hi16, klo_ref[...],
                                 jnp.int16(-32768))
        tb_lo = _bisect16(mlo_ref, ones_bf, kk_r - base_b, per_row=True)
        tb_ref[...] = lax.bitcast_convert_type((tb_hi << 16) | tb_lo,
                                               jnp.uint32)

    @pl.when(i >= _NB)
    def _():
        j = i - _NB
        sl = pl.ds(j * _BLK, _BLK)
        hi = khi_ref[:, sl].astype(jnp.int32) + 32768
        lo = klo_ref[:, sl].astype(jnp.int32) + 32768
        hi2 = khi2_ref[:, sl].astype(jnp.int32) + 32768
        k32 = lax.bitcast_convert_type((hi << 16) | lo, jnp.uint32)
        k232 = lax.bitcast_convert_type((hi2 << 16) | lo, jnp.uint32)
        lin_blk = _unkey(k32)
        main_m = k32 >= tg_ref[0]
        dead_m = k232 >= tb_ref[...]
        mvals = jnp.where(main_m, lin_blk, 0.0).astype(jnp.bfloat16)
        dvals = jnp.where(dead_m, lin_blk, 0.0).astype(jnp.bfloat16)
        wb = w_ref[...].astype(jnp.bfloat16)
        r = lax.dot_general(mvals, wb, (((1,), (0,)), ((), ())),
                            preferred_element_type=jnp.float32)
        d = lax.dot_general(dvals, wb, (((1,), (0,)), ((), ())),
                            preferred_element_type=jnp.float32)

        @pl.when(j == 0)
        def _():
            recon_ref[...] = r
            dead_ref[...] = d

        @pl.when(j > 0)
        def _():
            recon_ref[...] += r
            dead_ref[...] += d

        @pl.when(j == _NB - 1)
        def _():
            recon_ref[...] += bias_ref[...]


@jax.jit
def kernel(x, W_enc, W_dec, bias_pre, last_activation):
    del W_dec
    bias2d = bias_pre.reshape(1, _INPUT_DIM)
    la2d = last_activation.reshape(1, _SPARSE_DIM)

    recon, dead_recon = pl.pallas_call(
        _sae_kernel,
        grid=(2 * _NB,),
        in_specs=[
            pl.BlockSpec((_BATCH, _INPUT_DIM), lambda i: (0, 0)),
            pl.BlockSpec((_BLK, _INPUT_DIM), lambda i: (i % _NB, 0)),
            pl.BlockSpec((1, _INPUT_DIM), lambda i: (0, 0)),
            pl.BlockSpec((1, _SPARSE_DIM), lambda i: (0, 0)),
        ],
        out_specs=[
            pl.BlockSpec((_BATCH, _INPUT_DIM), lambda i: (0, 0)),
            pl.BlockSpec((_BATCH, _INPUT_DIM), lambda i: (0, 0)),
        ],
        out_shape=[
            jax.ShapeDtypeStruct((_BATCH, _INPUT_DIM), jnp.float32),
            jax.ShapeDtypeStruct((_BATCH, _INPUT_DIM), jnp.float32),
        ],
        scratch_shapes=[
            pltpu.VMEM((_BATCH, _SPARSE_DIM), jnp.int16),
            pltpu.VMEM((_BATCH, _SPARSE_DIM), jnp.int16),
            pltpu.VMEM((_BATCH, _SPARSE_DIM), jnp.int16),
            pltpu.VMEM((_BATCH, _SPARSE_DIM), jnp.int16),
            pltpu.VMEM((_BATCH, _INPUT_DIM), jnp.float32),
            pltpu.SMEM((1,), jnp.uint32),
            pltpu.VMEM((_BATCH, 1), jnp.uint32),
        ],
        compiler_params=pltpu.CompilerParams(
            dimension_semantics=("arbitrary",),
        ),
    )(x, W_enc, bias2d, la2d)
    return recon, dead_recon
